# baseline (speedup 1.0000x reference)
import jax
import jax.numpy as jnp
from jax import lax
from jax.experimental import pallas as pl
from jax.experimental.pallas import tpu as pltpu
from jax.experimental.pallas import tpu_sc as plsc

B = 16384
D = 32
T = 50

NC = 2
NS = 16
NW = NC * NS
BPW = B // NW
CH = 128
NCH = BPW // CH


def _sc_gather_body(u_hbm, i_hbm, ut32, it32, ul1, il1,
                    out_u, out_i, out_ul, out_il,
                    idx_u, idx_i, uvals, ivals, ulv, ilv, sem):
    wid = lax.axis_index("s") * NC + lax.axis_index("c")
    base = wid * BPW
    pltpu.sync_copy(u_hbm.at[pl.ds(base, BPW)], idx_u)
    pltpu.sync_copy(i_hbm.at[pl.ds(base, BPW)], idx_i)
    cps = []
    for c in range(NCH):
        sl = pl.ds(c * CH, CH)
        iu = idx_u.at[sl]
        ii = idx_i.at[sl]
        cps.append(pltpu.async_copy(ul1.at[iu], ulv.at[sl], sem))
        cps.append(pltpu.async_copy(il1.at[ii], ilv.at[sl], sem))
        for d in range(D):
            cps.append(pltpu.async_copy(ut32.at[d].at[iu],
                                        uvals.at[d, sl], sem))
            cps.append(pltpu.async_copy(it32.at[d].at[ii],
                                        ivals.at[d, sl], sem))
    for cp in cps:
        cp.wait()
    pltpu.sync_copy(uvals, out_u.at[:, pl.ds(base, BPW)])
    pltpu.sync_copy(ivals, out_i.at[:, pl.ds(base, BPW)])
    pltpu.sync_copy(ulv, out_ul.at[pl.ds(base, BPW)])
    pltpu.sync_copy(ilv, out_il.at[pl.ds(base, BPW)])


def _sc_gather(u, i, ut32, it32, ul1, il1):
    mesh = plsc.VectorSubcoreMesh(core_axis_name="c", subcore_axis_name="s",
                                  num_cores=NC, num_subcores=NS)
    f = pl.kernel(
        _sc_gather_body,
        out_type=[
            jax.ShapeDtypeStruct((D, B), jnp.float32),
            jax.ShapeDtypeStruct((D, B), jnp.float32),
            jax.ShapeDtypeStruct((B,), jnp.float32),
            jax.ShapeDtypeStruct((B,), jnp.float32),
        ],
        mesh=mesh,
        scratch_types=[
            pltpu.VMEM((BPW,), jnp.int32),
            pltpu.VMEM((BPW,), jnp.int32),
            pltpu.VMEM((D, BPW), jnp.float32),
            pltpu.VMEM((D, BPW), jnp.float32),
            pltpu.VMEM((BPW,), jnp.float32),
            pltpu.VMEM((BPW,), jnp.float32),
            pltpu.SemaphoreType.DMA,
        ],
        compiler_params=pltpu.CompilerParams(use_tc_tiling_on_sc=False),
    )
    return f(u, i, ut32, it32, ul1, il1)


def _dense_body(u_ref, i_ref, tf_ref, ul_ref, il_ref,
                tW_ref, tb_ref, tlw_ref,
                w1u_ref, w1i_ref, w1t_ref, b1_ref,
                w2_ref, b2_ref, w3_ref, sb_ref, out_ref):
    f32 = jnp.float32
    u = u_ref[...]
    it = i_ref[...]
    tf = tf_ref[...]
    t = jnp.dot(tW_ref[...], tf, preferred_element_type=f32) + tb_ref[...]
    fm2 = jnp.sum(u * it + (u + it) * t, axis=0)
    t_lin = jnp.sum(tf * tlw_ref[...], axis=0)
    fm1 = ul_ref[...] + il_ref[...] + t_lin
    h = jnp.dot(w1u_ref[...], u, preferred_element_type=f32)
    h += jnp.dot(w1i_ref[...], it, preferred_element_type=f32)
    h += jnp.dot(w1t_ref[...], t, preferred_element_type=f32)
    h = jnp.maximum(h + b1_ref[...], 0.0)
    h = jnp.maximum(jnp.dot(w2_ref[...], h, preferred_element_type=f32)
                    + b2_ref[...], 0.0)
    deep = jnp.sum(h * w3_ref[...], axis=0)
    z = fm1 + fm2 + deep + sb_ref[0, 0]
    out_ref[...] = jax.nn.sigmoid(z)


def _dense(u_embT, i_embT, tfT, u_lin, i_lin,
           tWT, tb, tlw, w1uT, w1iT, w1tT, b1, w2T, b2, w3, sb):
    bB = 2048
    grid = (B // bB,)
    col = lambda b: (0, b)
    rep = lambda b: (0, 0)
    vec = lambda b: (b,)
    return pl.pallas_call(
        _dense_body,
        grid=grid,
        in_specs=[
            pl.BlockSpec((D, bB), col),
            pl.BlockSpec((D, bB), col),
            pl.BlockSpec((T, bB), col),
            pl.BlockSpec((bB,), vec),
            pl.BlockSpec((bB,), vec),
            pl.BlockSpec((D, T), rep),
            pl.BlockSpec((D, 1), rep),
            pl.BlockSpec((T, 1), rep),
            pl.BlockSpec((64, D), rep),
            pl.BlockSpec((64, D), rep),
            pl.BlockSpec((64, D), rep),
            pl.BlockSpec((64, 1), rep),
            pl.BlockSpec((D, 64), rep),
            pl.BlockSpec((D, 1), rep),
            pl.BlockSpec((D, 1), rep),
            pl.BlockSpec((1, 1), rep),
        ],
        out_specs=pl.BlockSpec((bB,), vec),
        out_shape=jax.ShapeDtypeStruct((B,), jnp.float32),
    )(u_embT, i_embT, tfT, u_lin, i_lin, tWT, tb, tlw,
      w1uT, w1iT, w1tT, b1, w2T, b2, w3, sb)


def kernel(u, i, text_features, user_table, item_table, text_W, text_b,
           user_lin_table, item_lin_table, textlin_W, textlin_b, fm_bias,
           W1, b1, W2, b2, W3, b3):
    u = u.astype(jnp.int32)
    i = i.astype(jnp.int32)
    u_embT, i_embT, u_lin, i_lin = _sc_gather(
        u, i, user_table.T, item_table.T,
        user_lin_table.reshape(-1), item_lin_table.reshape(-1))
    if True:
        t = text_features @ text_W + text_b
        ue = u_embT.T
        ie = i_embT.T
        fm2 = jnp.sum(ue * ie + (ue + ie) * t, axis=1)
        fm1 = u_lin + i_lin + text_features @ textlin_W[:, 0]
        h = jax.nn.relu(ue @ W1[:D] + ie @ W1[D:2*D] + t @ W1[2*D:] + b1)
        h = jax.nn.relu(h @ W2 + b2)
        deep = h @ W3[:, 0]
        return jax.nn.sigmoid(fm1 + fm2 + deep + (fm_bias + textlin_b + b3)[0])
    sb = (fm_bias + textlin_b + b3).reshape(1, 1)
    return _dense(u_embT, i_embT, text_features.T, u_lin, i_lin,
                  text_W.T, text_b.reshape(D, 1), textlin_W.reshape(T, 1),
                  W1[:D].T, W1[D:2 * D].T, W1[2 * D:].T, b1.reshape(64, 1),
                  W2.T, b2.reshape(D, 1), W3.reshape(1, D).T, sb)


# speedup vs baseline: 1.0371x; 1.0371x over previous
import jax
import jax.numpy as jnp
from jax import lax
from jax.experimental import pallas as pl
from jax.experimental.pallas import tpu as pltpu
from jax.experimental.pallas import tpu_sc as plsc

B = 16384
D = 32
T = 50

NC = 2
NS = 16
NW = NC * NS
BPW = B // NW
CH = 128
NCH = BPW // CH


def _sc_gather_body(u_hbm, i_hbm, ut32, it32, ul1, il1,
                    out_u, out_i, out_ul, out_il,
                    idx_u, idx_i, uvals, ivals, ulv, ilv, sem):
    wid = lax.axis_index("s") * NC + lax.axis_index("c")
    base = wid * BPW
    pltpu.sync_copy(u_hbm.at[pl.ds(base, BPW)], idx_u)
    pltpu.sync_copy(i_hbm.at[pl.ds(base, BPW)], idx_i)
    cps = []
    for c in range(NCH):
        sl = pl.ds(c * CH, CH)
        cps.append(pltpu.async_copy(ul1.at[idx_u.at[sl]], ulv.at[sl], sem))
        cps.append(pltpu.async_copy(il1.at[idx_i.at[sl]], ilv.at[sl], sem))

    def _dim_step(d, carry):
        for c in range(NCH):
            sl = pl.ds(c * CH, CH)
            pltpu.async_copy(ut32.at[d].at[idx_u.at[sl]],
                             uvals.at[d, sl], sem)
            pltpu.async_copy(it32.at[d].at[idx_i.at[sl]],
                             ivals.at[d, sl], sem)
        return carry

    lax.fori_loop(0, D, _dim_step, 0)
    pltpu.make_async_copy(out_u.at[:, pl.ds(0, BPW)], uvals, sem).wait()
    pltpu.make_async_copy(out_i.at[:, pl.ds(0, BPW)], ivals, sem).wait()
    for cp in cps:
        cp.wait()
    pltpu.sync_copy(uvals, out_u.at[:, pl.ds(base, BPW)])
    pltpu.sync_copy(ivals, out_i.at[:, pl.ds(base, BPW)])
    pltpu.sync_copy(ulv, out_ul.at[pl.ds(base, BPW)])
    pltpu.sync_copy(ilv, out_il.at[pl.ds(base, BPW)])


def _sc_gather(u, i, ut32, it32, ul1, il1):
    mesh = plsc.VectorSubcoreMesh(core_axis_name="c", subcore_axis_name="s",
                                  num_cores=NC, num_subcores=NS)
    f = pl.kernel(
        _sc_gather_body,
        out_type=[
            jax.ShapeDtypeStruct((D, B), jnp.float32),
            jax.ShapeDtypeStruct((D, B), jnp.float32),
            jax.ShapeDtypeStruct((B,), jnp.float32),
            jax.ShapeDtypeStruct((B,), jnp.float32),
        ],
        mesh=mesh,
        scratch_types=[
            pltpu.VMEM((BPW,), jnp.int32),
            pltpu.VMEM((BPW,), jnp.int32),
            pltpu.VMEM((D, BPW), jnp.float32),
            pltpu.VMEM((D, BPW), jnp.float32),
            pltpu.VMEM((BPW,), jnp.float32),
            pltpu.VMEM((BPW,), jnp.float32),
            pltpu.SemaphoreType.DMA,
        ],
        compiler_params=pltpu.CompilerParams(use_tc_tiling_on_sc=False),
    )
    return f(u, i, ut32, it32, ul1, il1)


def _dense_body(u_ref, i_ref, tf_ref, ul_ref, il_ref,
                tW_ref, tb_ref, tlw_ref,
                w1u_ref, w1i_ref, w1t_ref, b1_ref,
                w2_ref, b2_ref, w3_ref, sb_ref, out_ref):
    f32 = jnp.float32
    u = u_ref[...]
    it = i_ref[...]
    tf = tf_ref[...]
    t = jnp.dot(tW_ref[...], tf, preferred_element_type=f32) + tb_ref[...]
    fm2 = jnp.sum(u * it + (u + it) * t, axis=0)
    t_lin = jnp.sum(tf * tlw_ref[...], axis=0)
    fm1 = ul_ref[...] + il_ref[...] + t_lin
    h = jnp.dot(w1u_ref[...], u, preferred_element_type=f32)
    h += jnp.dot(w1i_ref[...], it, preferred_element_type=f32)
    h += jnp.dot(w1t_ref[...], t, preferred_element_type=f32)
    h = jnp.maximum(h + b1_ref[...], 0.0)
    h = jnp.maximum(jnp.dot(w2_ref[...], h, preferred_element_type=f32)
                    + b2_ref[...], 0.0)
    deep = jnp.sum(h * w3_ref[...], axis=0)
    z = fm1 + fm2 + deep + sb_ref[0, 0]
    out_ref[...] = jax.nn.sigmoid(z)


def _dense(u_embT, i_embT, tfT, u_lin, i_lin,
           tWT, tb, tlw, w1uT, w1iT, w1tT, b1, w2T, b2, w3, sb):
    bB = 2048
    grid = (B // bB,)
    col = lambda b: (0, b)
    rep = lambda b: (0, 0)
    vec = lambda b: (b,)
    return pl.pallas_call(
        _dense_body,
        grid=grid,
        in_specs=[
            pl.BlockSpec((D, bB), col),
            pl.BlockSpec((D, bB), col),
            pl.BlockSpec((T, bB), col),
            pl.BlockSpec((bB,), vec),
            pl.BlockSpec((bB,), vec),
            pl.BlockSpec((D, T), rep),
            pl.BlockSpec((D, 1), rep),
            pl.BlockSpec((T, 1), rep),
            pl.BlockSpec((64, D), rep),
            pl.BlockSpec((64, D), rep),
            pl.BlockSpec((64, D), rep),
            pl.BlockSpec((64, 1), rep),
            pl.BlockSpec((D, 64), rep),
            pl.BlockSpec((D, 1), rep),
            pl.BlockSpec((D, 1), rep),
            pl.BlockSpec((1, 1), rep),
        ],
        out_specs=pl.BlockSpec((bB,), vec),
        out_shape=jax.ShapeDtypeStruct((B,), jnp.float32),
    )(u_embT, i_embT, tfT, u_lin, i_lin, tWT, tb, tlw,
      w1uT, w1iT, w1tT, b1, w2T, b2, w3, sb)


def kernel(u, i, text_features, user_table, item_table, text_W, text_b,
           user_lin_table, item_lin_table, textlin_W, textlin_b, fm_bias,
           W1, b1, W2, b2, W3, b3):
    u = u.astype(jnp.int32)
    i = i.astype(jnp.int32)
    u_embT, i_embT, u_lin, i_lin = _sc_gather(
        u, i, user_table.T, item_table.T,
        user_lin_table.reshape(-1), item_lin_table.reshape(-1))
    sb = (fm_bias + textlin_b + b3).reshape(1, 1)
    return _dense(u_embT, i_embT, text_features.T, u_lin, i_lin,
                  text_W.T, text_b.reshape(D, 1), textlin_W.reshape(T, 1),
                  W1[:D].T, W1[D:2 * D].T, W1[2 * D:].T, b1.reshape(64, 1),
                  W2.T, b2.reshape(D, 1), W3.reshape(1, D).T, sb)


# speedup vs baseline: 5.3195x; 5.1291x over previous
import jax
import jax.numpy as jnp
from jax import lax
from jax.experimental import pallas as pl
from jax.experimental.pallas import tpu as pltpu
from jax.experimental.pallas import tpu_sc as plsc

B = 16384
D = 32
T = 50
G = 4
W = 128
R = 1000000 // G

NC = 2
NS = 16
NW = NC * NS
BPW = B // NW
CH = 128
NCH = BPW // CH


def _sc_gather_body(us_hbm, is_hbm, u_hbm, i_hbm, rm_u, rm_i, ul1, il1,
                    out_u, out_i, out_ul, out_il,
                    idx_us, idx_is, idx_u, idx_i, slab, ulv, ilv, sem):
    wid = lax.axis_index("s") * NC + lax.axis_index("c")
    base = wid * BPW
    pltpu.sync_copy(us_hbm.at[pl.ds(base, BPW)], idx_us)
    pltpu.sync_copy(is_hbm.at[pl.ds(base, BPW)], idx_is)
    pltpu.sync_copy(u_hbm.at[pl.ds(base, BPW)], idx_u)
    pltpu.sync_copy(i_hbm.at[pl.ds(base, BPW)], idx_i)
    lins = []
    for c in range(NCH):
        sl = pl.ds(c * CH, CH)
        lins.append(pltpu.async_copy(ul1.at[idx_u.at[sl]], ulv.at[sl], sem))
        lins.append(pltpu.async_copy(il1.at[idx_i.at[sl]], ilv.at[sl], sem))
    for tab, idxv, out in ((rm_u, idx_us, out_u), (rm_i, idx_is, out_i)):
        cps = []
        for c in range(NCH):
            sl = pl.ds(c * CH, CH)
            cps.append(pltpu.async_copy(tab.at[idxv.at[sl]],
                                        slab.at[sl], sem))
        for cp in cps:
            cp.wait()
        pltpu.sync_copy(slab, out.at[pl.ds(base, BPW)])
    for cp in lins:
        cp.wait()
    pltpu.sync_copy(ulv, out_ul.at[pl.ds(base, BPW)])
    pltpu.sync_copy(ilv, out_il.at[pl.ds(base, BPW)])


def _sc_gather(us, is_, u, i, rm_u, rm_i, ul1, il1):
    mesh = plsc.VectorSubcoreMesh(core_axis_name="c", subcore_axis_name="s",
                                  num_cores=NC, num_subcores=NS)
    f = pl.kernel(
        _sc_gather_body,
        out_type=[
            jax.ShapeDtypeStruct((B, W), jnp.float32),
            jax.ShapeDtypeStruct((B, W), jnp.float32),
            jax.ShapeDtypeStruct((B,), jnp.float32),
            jax.ShapeDtypeStruct((B,), jnp.float32),
        ],
        mesh=mesh,
        scratch_types=[
            pltpu.VMEM((BPW,), jnp.int32),
            pltpu.VMEM((BPW,), jnp.int32),
            pltpu.VMEM((BPW,), jnp.int32),
            pltpu.VMEM((BPW,), jnp.int32),
            pltpu.VMEM((BPW, W), jnp.float32),
            pltpu.VMEM((BPW,), jnp.float32),
            pltpu.VMEM((BPW,), jnp.float32),
            pltpu.SemaphoreType.DMA,
        ],
        compiler_params=pltpu.CompilerParams(use_tc_tiling_on_sc=False),
    )
    return f(us, is_, u, i, rm_u, rm_i, ul1, il1)


def _dense_body(us_ref, is_ref, uoh_ref, ioh_ref, tf_ref, ul_ref, il_ref,
                tW_ref, tb_ref, tlw_ref,
                w1u_ref, w1i_ref, w1t_ref, b1_ref,
                w2_ref, b2_ref, w3_ref, sb_ref, out_ref):
    f32 = jnp.float32
    bB = us_ref.shape[0]
    uslab = us_ref[...].reshape(bB, G, D)
    islab = is_ref[...].reshape(bB, G, D)
    u = jnp.sum(uslab * uoh_ref[...][:, :, None], axis=1)
    it = jnp.sum(islab * ioh_ref[...][:, :, None], axis=1)
    tf = tf_ref[...]
    t = jnp.dot(tf, tW_ref[...], preferred_element_type=f32) + tb_ref[...]
    fm2 = jnp.sum(u * it + (u + it) * t, axis=1)
    t_lin = jnp.sum(tf * tlw_ref[...], axis=1)
    fm1 = ul_ref[...] + il_ref[...] + t_lin
    h = jnp.dot(u, w1u_ref[...], preferred_element_type=f32)
    h += jnp.dot(it, w1i_ref[...], preferred_element_type=f32)
    h += jnp.dot(t, w1t_ref[...], preferred_element_type=f32)
    h = jnp.maximum(h + b1_ref[...], 0.0)
    h = jnp.maximum(jnp.dot(h, w2_ref[...], preferred_element_type=f32)
                    + b2_ref[...], 0.0)
    deep = jnp.sum(h * w3_ref[...], axis=1)
    z = fm1 + fm2 + deep + sb_ref[0, 0]
    out_ref[...] = jax.nn.sigmoid(z)


def _dense(uslab, islab, uoh, ioh, tf, u_lin, i_lin,
           tW, tb, tlw, w1u, w1i, w1t, b1, w2, b2, w3, sb):
    bB = 2048
    grid = (B // bB,)
    row = lambda b: (b, 0)
    rep = lambda b: (0, 0)
    vec = lambda b: (b,)
    return pl.pallas_call(
        _dense_body,
        grid=grid,
        in_specs=[
            pl.BlockSpec((bB, W), row),
            pl.BlockSpec((bB, W), row),
            pl.BlockSpec((bB, G), row),
            pl.BlockSpec((bB, G), row),
            pl.BlockSpec((bB, T), row),
            pl.BlockSpec((bB,), vec),
            pl.BlockSpec((bB,), vec),
            pl.BlockSpec((T, D), rep),
            pl.BlockSpec((1, D), rep),
            pl.BlockSpec((1, T), rep),
            pl.BlockSpec((D, 64), rep),
            pl.BlockSpec((D, 64), rep),
            pl.BlockSpec((D, 64), rep),
            pl.BlockSpec((1, 64), rep),
            pl.BlockSpec((64, D), rep),
            pl.BlockSpec((1, D), rep),
            pl.BlockSpec((1, D), rep),
            pl.BlockSpec((1, 1), rep),
        ],
        out_specs=pl.BlockSpec((bB,), vec),
        out_shape=jax.ShapeDtypeStruct((B,), jnp.float32),
    )(uslab, islab, uoh, ioh, tf, u_lin, i_lin, tW, tb, tlw,
      w1u, w1i, w1t, b1, w2, b2, w3, sb)


def kernel(u, i, text_features, user_table, item_table, text_W, text_b,
           user_lin_table, item_lin_table, textlin_W, textlin_b, fm_bias,
           W1, b1, W2, b2, W3, b3):
    u = u.astype(jnp.int32)
    i = i.astype(jnp.int32)
    rm_u = user_table.reshape(R, W)
    rm_i = item_table.reshape(R, W)
    uslab, islab, u_lin, i_lin = _sc_gather(
        u >> 2, i >> 2, u, i, rm_u, rm_i,
        user_lin_table.reshape(-1), item_lin_table.reshape(-1))
    grp = jnp.arange(G, dtype=jnp.int32)[None, :]
    uoh = ((u & (G - 1))[:, None] == grp).astype(jnp.float32)
    ioh = ((i & (G - 1))[:, None] == grp).astype(jnp.float32)
    sb = (fm_bias + textlin_b + b3).reshape(1, 1)
    return _dense(uslab, islab, uoh, ioh, text_features, u_lin, i_lin,
                  text_W, text_b.reshape(1, D), textlin_W.reshape(1, T),
                  W1[:D], W1[D:2 * D], W1[2 * D:], b1.reshape(1, 64),
                  W2, b2.reshape(1, D), W3.reshape(1, D), sb)


# speedup vs baseline: 5.7163x; 1.0746x over previous
import jax
import jax.numpy as jnp
from jax import lax
from jax.experimental import pallas as pl
from jax.experimental.pallas import tpu as pltpu
from jax.experimental.pallas import tpu_sc as plsc

B = 16384
D = 32
T = 50
L = 16

NC = 2
NS = 16
NW = NC * NS
BPW = B // NW
CH = 128
NCH = BPW // CH


def _sc_gather_body(u_hbm, i_hbm, us_hbm, is_hbm, ut, it, ult16, ilt16,
                    out_u, out_i, out_ul, out_il,
                    idx_u, idx_i, sidx_u, sidx_i,
                    urows, irows, ulrows, ilrows, sem):
    wid = lax.axis_index("s") * NC + lax.axis_index("c")
    base = wid * BPW
    pltpu.sync_copy(u_hbm.at[pl.ds(base, BPW)], idx_u)
    pltpu.sync_copy(i_hbm.at[pl.ds(base, BPW)], idx_i)
    pltpu.sync_copy(us_hbm.at[pl.ds(base, BPW)], sidx_u)
    pltpu.sync_copy(is_hbm.at[pl.ds(base, BPW)], sidx_i)
    cps = []
    for c in range(NCH):
        sl = pl.ds(c * CH, CH)
        cps.append(pltpu.async_copy(ut.at[idx_u.at[sl]], urows.at[sl], sem))
        cps.append(pltpu.async_copy(it.at[idx_i.at[sl]], irows.at[sl], sem))
        cps.append(pltpu.async_copy(ult16.at[sidx_u.at[sl]], ulrows.at[sl], sem))
        cps.append(pltpu.async_copy(ilt16.at[sidx_i.at[sl]], ilrows.at[sl], sem))
    for cp in cps:
        cp.wait()
    pltpu.sync_copy(urows, out_u.at[pl.ds(base, BPW)])
    pltpu.sync_copy(irows, out_i.at[pl.ds(base, BPW)])
    pltpu.sync_copy(ulrows, out_ul.at[pl.ds(base, BPW)])
    pltpu.sync_copy(ilrows, out_il.at[pl.ds(base, BPW)])


def _sc_gather(u, i, us, is_, user_table, item_table, ult16, ilt16):
    mesh = plsc.VectorSubcoreMesh(core_axis_name="c", subcore_axis_name="s",
                                  num_cores=NC, num_subcores=NS)
    f = pl.kernel(
        _sc_gather_body,
        out_type=[
            jax.ShapeDtypeStruct((B, D), jnp.float32),
            jax.ShapeDtypeStruct((B, D), jnp.float32),
            jax.ShapeDtypeStruct((B, L), jnp.float32),
            jax.ShapeDtypeStruct((B, L), jnp.float32),
        ],
        mesh=mesh,
        scratch_types=[
            pltpu.VMEM((BPW,), jnp.int32),
            pltpu.VMEM((BPW,), jnp.int32),
            pltpu.VMEM((BPW,), jnp.int32),
            pltpu.VMEM((BPW,), jnp.int32),
            pltpu.VMEM((BPW, D), jnp.float32),
            pltpu.VMEM((BPW, D), jnp.float32),
            pltpu.VMEM((BPW, L), jnp.float32),
            pltpu.VMEM((BPW, L), jnp.float32),
            pltpu.SemaphoreType.DMA,
        ],
        compiler_params=pltpu.CompilerParams(use_tc_tiling_on_sc=False),
    )
    return f(u, i, us, is_, user_table, item_table, ult16, ilt16)


def _dense_body(u_ref, i_ref, tf_ref, ul_ref, il_ref, uc_ref, ic_ref,
                tW_ref, tb_ref, tlw_ref,
                w1u_ref, w1i_ref, w1t_ref, b1_ref,
                w2_ref, b2_ref, w3_ref, sb_ref, out_ref):
    f32 = jnp.float32
    u = u_ref[...]
    it = i_ref[...]
    tf = tf_ref[...]
    t = jnp.dot(tf, tW_ref[...], preferred_element_type=f32) + tb_ref[...]
    fm2 = jnp.sum(u * it + (u + it) * t, axis=1)
    t_lin = jnp.sum(tf * tlw_ref[...], axis=1)
    lane = lax.broadcasted_iota(jnp.int32, (1, L), 1)
    u_lin = jnp.sum(jnp.where(uc_ref[...][:, None] == lane, ul_ref[...], 0.0),
                    axis=1)
    i_lin = jnp.sum(jnp.where(ic_ref[...][:, None] == lane, il_ref[...], 0.0),
                    axis=1)
    fm1 = u_lin + i_lin + t_lin
    h = jnp.dot(u, w1u_ref[...], preferred_element_type=f32)
    h += jnp.dot(it, w1i_ref[...], preferred_element_type=f32)
    h += jnp.dot(t, w1t_ref[...], preferred_element_type=f32)
    h = jnp.maximum(h + b1_ref[...], 0.0)
    h = jnp.maximum(jnp.dot(h, w2_ref[...], preferred_element_type=f32)
                    + b2_ref[...], 0.0)
    deep = jnp.sum(h * w3_ref[...], axis=1)
    z = fm1 + fm2 + deep + sb_ref[0, 0]
    out_ref[...] = jax.nn.sigmoid(z)


def _dense(u_emb, i_emb, tf, ul16, il16, ucol, icol,
           tW, tb, tlw, w1u, w1i, w1t, b1, w2, b2, w3, sb):
    bB = 2048
    grid = (B // bB,)
    row = lambda b: (b, 0)
    rep = lambda b: (0, 0)
    vec = lambda b: (b,)
    return pl.pallas_call(
        _dense_body,
        grid=grid,
        in_specs=[
            pl.BlockSpec((bB, D), row),
            pl.BlockSpec((bB, D), row),
            pl.BlockSpec((bB, T), row),
            pl.BlockSpec((bB, L), row),
            pl.BlockSpec((bB, L), row),
            pl.BlockSpec((bB,), vec),
            pl.BlockSpec((bB,), vec),
            pl.BlockSpec((T, D), rep),
            pl.BlockSpec((1, D), rep),
            pl.BlockSpec((1, T), rep),
            pl.BlockSpec((D, 64), rep),
            pl.BlockSpec((D, 64), rep),
            pl.BlockSpec((D, 64), rep),
            pl.BlockSpec((1, 64), rep),
            pl.BlockSpec((64, D), rep),
            pl.BlockSpec((1, D), rep),
            pl.BlockSpec((1, D), rep),
            pl.BlockSpec((1, 1), rep),
        ],
        out_specs=pl.BlockSpec((bB,), vec),
        out_shape=jax.ShapeDtypeStruct((B,), jnp.float32),
    )(u_emb, i_emb, tf, ul16, il16, ucol, icol, tW, tb, tlw,
      w1u, w1i, w1t, b1, w2, b2, w3, sb)


def kernel(u, i, text_features, user_table, item_table, text_W, text_b,
           user_lin_table, item_lin_table, textlin_W, textlin_b, fm_bias,
           W1, b1, W2, b2, W3, b3):
    u = u.astype(jnp.int32)
    i = i.astype(jnp.int32)
    ult16 = user_lin_table.reshape(-1, L)
    ilt16 = item_lin_table.reshape(-1, L)
    u_emb, i_emb, ul16, il16 = _sc_gather(u, i, u >> 4, i >> 4,
                                          user_table, item_table, ult16, ilt16)
    sb = (fm_bias + textlin_b + b3).reshape(1, 1)
    return _dense(u_emb, i_emb, text_features, ul16, il16, u & (L - 1),
                  i & (L - 1),
                  text_W, text_b.reshape(1, D), textlin_W.reshape(1, T),
                  W1[:D], W1[D:2 * D], W1[2 * D:], b1.reshape(1, 64),
                  W2, b2.reshape(1, D), W3.reshape(1, D), sb)
